# baseline (speedup 1.0000x reference)
import functools
import math

import jax
import jax.numpy as jnp
from jax.experimental import pallas as pl

_N = 10000
_E = 160000


def _mm_body(x_ref, w_ref, o_ref):
    o_ref[...] = jnp.dot(x_ref[...], w_ref[...],
                         preferred_element_type=jnp.float32)


@functools.partial(jax.jit, static_argnames=("bm",))
def _mm(x, w, bm=1000):
    m, k = x.shape
    n = w.shape[1]
    grid = (m // bm,)
    return pl.pallas_call(
        _mm_body,
        grid=grid,
        in_specs=[
            pl.BlockSpec((bm, k), lambda i: (i, 0)),
            pl.BlockSpec((k, n), lambda i: (0, 0)),
        ],
        out_specs=pl.BlockSpec((bm, n), lambda i: (i, 0)),
        out_shape=jax.ShapeDtypeStruct((m, n), jnp.float32),
    )(x, w)


def _gcn_phase(x, src_c, dst_c, norm, W, b, n):
    h = _mm(x, W)
    out = jax.ops.segment_sum(h[src_c] * norm[:, None], dst_c,
                              num_segments=n)
    return jax.nn.relu(out + b)


def kernel(x, edge_index, W1, b1, W2, b2, W3, b3, p1, W4, b4, W5, b5, W6,
           b6, p2, Wlin, blin):
    src = edge_index[0]
    dst = edge_index[1]
    f32 = jnp.float32

    loop = jnp.arange(_N, dtype=src.dtype)
    src_c = jnp.concatenate([src, loop])
    dst_c = jnp.concatenate([dst, loop])
    w_c = jnp.concatenate([jnp.ones((_E,), f32), jnp.ones((_N,), f32)])
    deg = jax.ops.segment_sum(w_c, dst_c, num_segments=_N)
    dinv = jnp.where(deg > 0, 1.0 / jnp.sqrt(deg), 0.0)
    norm = dinv[src_c] * dinv[dst_c] * w_c

    h1 = _gcn_phase(x, src_c, dst_c, norm, W1, b1, _N)
    h2 = _gcn_phase(h1, src_c, dst_c, norm, W2, b2, _N)
    h3 = _gcn_phase(h2, src_c, dst_c, norm, W3, b3, _N)

    s1 = jnp.tanh((h3 @ p1) / jnp.linalg.norm(p1))
    k1 = int(math.ceil(0.9 * _N))
    vals1, perm1 = jax.lax.top_k(s1, k1)
    xc = h3[perm1] * vals1[:, None]
    mapping = jnp.full((_N,), -1, jnp.int32).at[perm1].set(
        jnp.arange(k1, dtype=jnp.int32))
    ms = mapping[src]
    md = mapping[dst]
    valid = (ms >= 0) & (md >= 0)
    loop2 = jnp.arange(k1, dtype=src.dtype)
    src2_c = jnp.concatenate([jnp.where(valid, ms, 0), loop2])
    dst2_c = jnp.concatenate([jnp.where(valid, md, 0), loop2])
    w2_c = jnp.concatenate([valid.astype(f32), jnp.ones((k1,), f32)])
    deg2 = jax.ops.segment_sum(w2_c, dst2_c, num_segments=k1)
    dinv2 = jnp.where(deg2 > 0, 1.0 / jnp.sqrt(deg2), 0.0)
    norm2 = dinv2[src2_c] * dinv2[dst2_c] * w2_c

    h4 = _gcn_phase(xc, src2_c, dst2_c, norm2, W4, b4, k1)
    h5 = _gcn_phase(h4, src2_c, dst2_c, norm2, W5, b5, k1)
    h6 = _gcn_phase(h5, src2_c, dst2_c, norm2, W6, b6, k1)

    s2 = jnp.tanh((h6 @ p2) / jnp.linalg.norm(p2))
    k2 = int(math.ceil(0.9 * k1))
    vals2, perm2 = jax.lax.top_k(s2, k2)

    g = jnp.mean(h6[perm2] * vals2[:, None], axis=0, keepdims=True)
    logits = g @ Wlin + blin
    return (jax.nn.log_softmax(logits, axis=1), perm1, perm2)


# speedup vs baseline: 1.0000x; 1.0000x over previous
import functools
import math

import jax
import jax.numpy as jnp
from jax.experimental import pallas as pl

_N = 10000
_E = 160000


def _mm_body(x_ref, w_ref, o_ref):
    o_ref[...] = jnp.dot(x_ref[...], w_ref[...],
                         preferred_element_type=jnp.float32)


@functools.partial(jax.jit, static_argnames=("bm",))
def _mm(x, w, bm=1000):
    m, k = x.shape
    n = w.shape[1]
    return pl.pallas_call(
        _mm_body,
        grid=(m // bm,),
        in_specs=[
            pl.BlockSpec((bm, k), lambda i: (i, 0)),
            pl.BlockSpec((k, n), lambda i: (0, 0)),
        ],
        out_specs=pl.BlockSpec((bm, n), lambda i: (i, 0)),
        out_shape=jax.ShapeDtypeStruct((m, n), jnp.float32),
    )(x, w)


def _gcn_phase(x, src_c, dst_c, norm, W, b, n):
    h = _mm(x, W)
    out = jax.ops.segment_sum(h[src_c] * norm[:, None], dst_c,
                              num_segments=n)
    return jax.nn.relu(out + b)


def kernel(x, edge_index, W1, b1, W2, b2, W3, b3, p1, W4, b4, W5, b5, W6,
           b6, p2, Wlin, blin):
    src = edge_index[0]
    dst = edge_index[1]
    f32 = jnp.float32

    loop = jnp.arange(_N, dtype=src.dtype)
    src_c = jnp.concatenate([src, loop])
    dst_c = jnp.concatenate([dst, loop])
    w_c = jnp.concatenate([jnp.ones((_E,), f32), jnp.ones((_N,), f32)])
    deg = jax.ops.segment_sum(w_c, dst_c, num_segments=_N)
    dinv = jnp.where(deg > 0, 1.0 / jnp.sqrt(deg), 0.0)
    norm = dinv[src_c] * dinv[dst_c] * w_c

    h1 = _gcn_phase(x, src_c, dst_c, norm, W1, b1, _N)
    h2 = _gcn_phase(h1, src_c, dst_c, norm, W2, b2, _N)
    h3 = _gcn_phase(h2, src_c, dst_c, norm, W3, b3, _N)

    s1 = jnp.tanh((h3 @ p1) / jnp.linalg.norm(p1))
    k1 = int(math.ceil(0.9 * _N))
    vals1, perm1 = jax.lax.top_k(s1, k1)
    xc = h3[perm1] * vals1[:, None]
    mapping = jnp.full((_N,), -1, jnp.int32).at[perm1].set(
        jnp.arange(k1, dtype=jnp.int32))
    ms = mapping[src]
    md = mapping[dst]
    valid = (ms >= 0) & (md >= 0)
    loop2 = jnp.arange(k1, dtype=src.dtype)
    src2_c = jnp.concatenate([jnp.where(valid, ms, 0), loop2])
    dst2_c = jnp.concatenate([jnp.where(valid, md, 0), loop2])
    w2_c = jnp.concatenate([valid.astype(f32), jnp.ones((k1,), f32)])
    deg2 = jax.ops.segment_sum(w2_c, dst2_c, num_segments=k1)
    dinv2 = jnp.where(deg2 > 0, 1.0 / jnp.sqrt(deg2), 0.0)
    norm2 = dinv2[src2_c] * dinv2[dst2_c] * w2_c

    h4 = _gcn_phase(xc, src2_c, dst2_c, norm2, W4, b4, k1)
    h5 = _gcn_phase(h4, src2_c, dst2_c, norm2, W5, b5, k1)
    h6 = _gcn_phase(h5, src2_c, dst2_c, norm2, W6, b6, k1)

    s2 = jnp.tanh((h6 @ p2) / jnp.linalg.norm(p2))
    k2 = int(math.ceil(0.9 * k1))
    vals2, perm2 = jax.lax.top_k(s2, k2)

    g = jnp.mean(h6[perm2] * vals2[:, None], axis=0, keepdims=True)
    logits = g @ Wlin + blin
    return (jax.nn.log_softmax(logits, axis=1), perm1, perm2)
